# Initial kernel scaffold; baseline (speedup 1.0000x reference)
import jax
import jax.numpy as jnp
from jax.experimental import pallas as pl


def kernel(h, edge_index, id, e, W_emb, b_emb, W_gcn, b_gcn, gamma, beta, W0, b0, W1, b1, W2, b2):
    raise NotImplementedError("write your pallas kernel here")


# speedup vs baseline: 6.7393x; 6.7393x over previous
import functools
import jax
import jax.numpy as jnp
from jax import lax
from jax.experimental import pallas as pl
from jax.experimental.pallas import tpu as pltpu, tpu_sc as plsc

N = 50000
E = 800000
IN_DIM = 512
HID = 32
NUM_GRAPHS = 100
NODES_PER_GRAPH = 500

NPAD = 50176
EPAD = 819200
NC, NS = 2, 16
NW = NC * NS
EPT = EPAD // NW
CH = 512
CROWS = CH // 128
NCHUNK = EPT // CH
HCH = 1024
HEPT = EPAD // NS
HNCHUNK = HEPT // HCH
HWID = 16
ROWS_PT = NPAD // NS

_mesh = plsc.VectorSubcoreMesh(core_axis_name="c", subcore_axis_name="s")
_sc_params = pltpu.CompilerParams(use_tc_tiling_on_sc=False)


@functools.partial(
    pl.kernel,
    out_type=jax.ShapeDtypeStruct((NC, NPAD, HWID), jnp.float32),
    mesh=_mesh,
    compiler_params=_sc_params,
    scratch_types=[
        pltpu.VMEM_SHARED((NPAD, HWID), jnp.float32),
        pltpu.VMEM((8, 128), jnp.int32),
        pltpu.VMEM((128, HWID), jnp.float32),
    ],
)
def _hist_call(src2d, dst2d, ones_hbm, zeros_hbm, hist_out, shared, idx_v, ones_v):
    c = lax.axis_index("c")
    s = lax.axis_index("s")
    pltpu.sync_copy(zeros_hbm.at[pl.ds(s * ROWS_PT, ROWS_PT)],
                    shared.at[pl.ds(s * ROWS_PT, ROWS_PT)])
    pltpu.sync_copy(ones_hbm, ones_v)
    plsc.subcore_barrier()

    def run(edge2d):
        base_r = s * (HEPT // 128)

        def chunk(k, carry):
            pltpu.sync_copy(edge2d.at[pl.ds(base_r + k * (HCH // 128),
                                            HCH // 128)], idx_v)
            for j in range(HCH // 128):
                pltpu.sync_copy(ones_v, shared.at[idx_v.at[j]], add=True)
            return carry

        lax.fori_loop(0, HNCHUNK, chunk, 0)

    @pl.when(c == 0)
    def _():
        run(src2d)

    @pl.when(c == 1)
    def _():
        run(dst2d)

    plsc.subcore_barrier()
    pltpu.sync_copy(shared.at[pl.ds(s * ROWS_PT, ROWS_PT)],
                    hist_out.at[c, pl.ds(s * ROWS_PT, ROWS_PT)])


def _dense_body(h_ref, hist_ref, wemb_ref, bemb_ref, wgcn_ref, h1_ref, m_ref):
    h1 = jnp.dot(h_ref[...], wemb_ref[...],
                 preferred_element_type=jnp.float32) + bemb_ref[...]
    h1_ref[...] = h1
    deg = hist_ref[...][:, 0:1]
    scale = lax.rsqrt(jnp.maximum(deg, 1.0))
    m_ref[...] = jnp.dot(h1, wgcn_ref[...],
                         preferred_element_type=jnp.float32) * scale


def _dense_call(h, hist0, W_emb, b_emb, W_gcn):
    blk = 512
    grid = NPAD // blk
    return pl.pallas_call(
        _dense_body,
        grid=(grid,),
        in_specs=[
            pl.BlockSpec((blk, IN_DIM), lambda i: (i, 0)),
            pl.BlockSpec((blk, HWID), lambda i: (i, 0)),
            pl.BlockSpec((IN_DIM, HID), lambda i: (0, 0)),
            pl.BlockSpec((1, HID), lambda i: (0, 0)),
            pl.BlockSpec((HID, HID), lambda i: (0, 0)),
        ],
        out_specs=[
            pl.BlockSpec((blk, HID), lambda i: (i, 0)),
            pl.BlockSpec((blk, HID), lambda i: (i, 0)),
        ],
        out_shape=[
            jax.ShapeDtypeStruct((NPAD, HID), jnp.float32),
            jax.ShapeDtypeStruct((NPAD, HID), jnp.float32),
        ],
    )(h, hist0, W_emb, b_emb, W_gcn)


@functools.partial(
    pl.kernel,
    out_type=jax.ShapeDtypeStruct((NC, NPAD, HID), jnp.float32),
    mesh=_mesh,
    compiler_params=_sc_params,
    scratch_types=[
        pltpu.VMEM_SHARED((NPAD, HID), jnp.float32),
        pltpu.VMEM((CROWS, 128), jnp.int32),
        pltpu.VMEM((CROWS, 128), jnp.int32),
        pltpu.VMEM((CH, HID), jnp.float32),
        pltpu.SemaphoreType.DMA,
    ],
)
def _gs_call(m_hbm, src2d, dst2d, zeros_hbm, agg_out,
             shared, src_v, dst_v, rows_v, sem):
    c = lax.axis_index("c")
    s = lax.axis_index("s")
    wid = s * NC + c
    pltpu.sync_copy(zeros_hbm.at[pl.ds(s * ROWS_PT, ROWS_PT)],
                    shared.at[pl.ds(s * ROWS_PT, ROWS_PT)])
    plsc.subcore_barrier()

    base_r = wid * (EPT // 128)

    def chunk(k, carry):
        r0 = base_r + k * CROWS
        pltpu.sync_copy(src2d.at[pl.ds(r0, CROWS)], src_v)
        pltpu.sync_copy(dst2d.at[pl.ds(r0, CROWS)], dst_v)
        cps = [pltpu.async_copy(m_hbm.at[src_v.at[j]],
                                rows_v.at[pl.ds(j * 128, 128)], sem)
               for j in range(CROWS)]
        for cp in cps:
            cp.wait()
        for j in range(CROWS):
            pltpu.sync_copy(rows_v.at[pl.ds(j * 128, 128)],
                            shared.at[dst_v.at[j]], add=True)
        return carry

    lax.fori_loop(0, NCHUNK, chunk, 0)

    plsc.subcore_barrier()
    pltpu.sync_copy(shared.at[pl.ds(s * ROWS_PT, ROWS_PT)],
                    agg_out.at[c, pl.ds(s * ROWS_PT, ROWS_PT)])


def _final_body(h1_ref, agg_ref, hist_ref, bg_ref, g_ref, beta_ref, hgp_ref):
    agg = agg_ref[0] + agg_ref[1]
    deg = hist_ref[...][:, 0:1]
    agg = agg * lax.rsqrt(jnp.maximum(deg, 1.0)) + bg_ref[...]
    xx = agg * g_ref[...] + beta_ref[...]
    xx = jnp.where(xx > 0, xx, 0.01 * xx)
    xx = h1_ref[...] + xx
    s0 = jnp.sum(xx[:NODES_PER_GRAPH], axis=0, keepdims=True)
    s1 = jnp.sum(xx[NODES_PER_GRAPH:], axis=0, keepdims=True)
    z = jnp.zeros((6, HID), jnp.float32)
    hgp_ref[...] = jnp.concatenate([s0, s1, z], axis=0) * (1.0 / NODES_PER_GRAPH)


def _final_call(h1p, aggp, hist1, bg, g, beta):
    blk = 2 * NODES_PER_GRAPH
    grid = N // blk
    return pl.pallas_call(
        _final_body,
        grid=(grid,),
        in_specs=[
            pl.BlockSpec((blk, HID), lambda i: (i, 0)),
            pl.BlockSpec((NC, blk, HID), lambda i: (0, i, 0)),
            pl.BlockSpec((blk, HWID), lambda i: (i, 0)),
            pl.BlockSpec((1, HID), lambda i: (0, 0)),
            pl.BlockSpec((1, HID), lambda i: (0, 0)),
            pl.BlockSpec((1, HID), lambda i: (0, 0)),
        ],
        out_specs=pl.BlockSpec((8, HID), lambda i: (i, 0)),
        out_shape=jax.ShapeDtypeStruct((400, HID), jnp.float32),
    )(h1p, aggp, hist1, bg, g, beta)


def _mlp_body(hg_ref, w0, b0, w1, b1, w2, b2, out_ref):
    y = jnp.maximum(jnp.dot(hg_ref[...], w0[...],
                            preferred_element_type=jnp.float32) + b0[...], 0.0)
    y = jnp.maximum(jnp.dot(y, w1[...],
                            preferred_element_type=jnp.float32) + b1[...], 0.0)
    out_ref[...] = jnp.dot(y, w2[...],
                           preferred_element_type=jnp.float32) + b2[...]


def _mlp_call(hg, W0, b0, W1, b1, W2, b2):
    return pl.pallas_call(
        _mlp_body,
        out_shape=jax.ShapeDtypeStruct((NUM_GRAPHS, 2), jnp.float32),
    )(hg, W0, b0, W1, b1, W2, b2)


def kernel(h, edge_index, id, e, W_emb, b_emb, W_gcn, b_gcn, gamma, beta,
           W0, b0, W1, b1, W2, b2):
    src = edge_index[0]
    dst = edge_index[1]
    pad = jnp.full((EPAD - E,), N, jnp.int32)
    src2d = jnp.concatenate([src, pad]).reshape(EPAD // 128, 128)
    dst2d = jnp.concatenate([dst, pad]).reshape(EPAD // 128, 128)
    ones_h = jnp.ones((128, HWID), jnp.float32)
    zeros_h = jnp.zeros((NPAD, HWID), jnp.float32)
    zeros_a = jnp.zeros((NPAD, HID), jnp.float32)

    hist = _hist_call(src2d, dst2d, ones_h, zeros_h)
    h1p, mp = _dense_call(h, hist[0], W_emb, b_emb.reshape(1, HID), W_gcn)
    aggp = _gs_call(mp, src2d, dst2d, zeros_a)

    g = (gamma / jnp.sqrt(1.0 + 1e-5)).reshape(1, HID)
    hgp = _final_call(h1p, aggp, hist[1], b_gcn.reshape(1, HID), g,
                      beta.reshape(1, HID))
    hg = hgp.reshape(50, 8, HID)[:, :2].reshape(NUM_GRAPHS, HID)
    return _mlp_call(hg, W0, b0.reshape(1, 16), W1, b1.reshape(1, 8),
                     W2, b2.reshape(1, 2))


# speedup vs baseline: 14.1449x; 2.0989x over previous
import functools
import jax
import jax.numpy as jnp
from jax import lax
from jax.experimental import pallas as pl
from jax.experimental.pallas import tpu as pltpu, tpu_sc as plsc

N = 50000
E = 800000
IN_DIM = 512
HID = 32
NUM_GRAPHS = 100
NODES_PER_GRAPH = 500

ER = E // 128
NC, NS = 2, 16
NW = NC * NS
S3_STEPS = 196
S1_ROWS = ER // NS
HWID = 16
ZPT = N // NS

_mesh = plsc.VectorSubcoreMesh(core_axis_name="c", subcore_axis_name="s")
_sc_params = pltpu.CompilerParams(use_tc_tiling_on_sc=False)


@functools.partial(
    pl.kernel,
    out_type=jax.ShapeDtypeStruct((NC, N, HWID), jnp.float32),
    mesh=_mesh,
    compiler_params=_sc_params,
    scratch_types=[
        pltpu.VMEM_SHARED((N, HWID), jnp.float32),
        pltpu.VMEM((S1_ROWS, 128), jnp.int32),
        pltpu.VMEM((1, 128), jnp.int32),
        pltpu.VMEM((128, HWID), jnp.float32),
        pltpu.SemaphoreType.DMA,
    ],
)
def _hist_call(src2d, dst2d, ones_hbm, zeros_hbm, hist_out,
               shared, idxs, idx_x, ones_v, sem):
    c = lax.axis_index("c")
    s = lax.axis_index("s")
    pltpu.sync_copy(zeros_hbm.at[pl.ds(s * ZPT, ZPT)],
                    shared.at[pl.ds(s * ZPT, ZPT)])
    pltpu.sync_copy(ones_hbm, ones_v)

    def run(edge2d):
        pltpu.sync_copy(edge2d.at[pl.ds(s * S1_ROWS, S1_ROWS)], idxs)

        @pl.when(s < ER - NS * S1_ROWS)
        def _():
            pltpu.sync_copy(edge2d.at[pl.ds(NS * S1_ROWS + s, 1)], idx_x)

        plsc.subcore_barrier()

        def fire(k, carry):
            pltpu.async_copy(ones_v, shared.at[idxs.at[k]], sem, add=True)
            return carry

        lax.fori_loop(0, S1_ROWS, fire, 0)

        @pl.when(s < ER - NS * S1_ROWS)
        def _():
            pltpu.async_copy(ones_v, shared.at[idx_x.at[0]], sem, add=True)

        def drain(k, carry):
            pltpu.make_async_copy(ones_hbm, ones_v, sem).wait()
            return carry

        lax.fori_loop(0, S1_ROWS, drain, 0)

        @pl.when(s < ER - NS * S1_ROWS)
        def _():
            pltpu.make_async_copy(ones_hbm, ones_v, sem).wait()

    @pl.when(c == 0)
    def _():
        run(src2d)

    @pl.when(c == 1)
    def _():
        run(dst2d)

    plsc.subcore_barrier()
    pltpu.sync_copy(shared.at[pl.ds(s * ZPT, ZPT)],
                    hist_out.at[c, pl.ds(s * ZPT, ZPT)])


def _dense_body(h_ref, hist_ref, wemb_ref, bemb_ref, wgcn_ref, h1_ref, m_ref):
    h1 = jnp.dot(h_ref[...], wemb_ref[...],
                 preferred_element_type=jnp.float32) + bemb_ref[...]
    h1_ref[...] = h1
    deg = hist_ref[0][:, 0:1]
    scale = lax.rsqrt(jnp.maximum(deg, 1.0))
    m_ref[...] = jnp.dot(h1, wgcn_ref[...],
                         preferred_element_type=jnp.float32) * scale


def _dense_call(h, hist, W_emb, b_emb, W_gcn):
    blk = 2000
    return pl.pallas_call(
        _dense_body,
        grid=(N // blk,),
        in_specs=[
            pl.BlockSpec((blk, IN_DIM), lambda i: (i, 0)),
            pl.BlockSpec((1, blk, HWID), lambda i: (0, i, 0)),
            pl.BlockSpec((IN_DIM, HID), lambda i: (0, 0)),
            pl.BlockSpec((1, HID), lambda i: (0, 0)),
            pl.BlockSpec((HID, HID), lambda i: (0, 0)),
        ],
        out_specs=[
            pl.BlockSpec((blk, HID), lambda i: (i, 0)),
            pl.BlockSpec((blk, HID), lambda i: (i, 0)),
        ],
        out_shape=[
            jax.ShapeDtypeStruct((N, HID), jnp.float32),
            jax.ShapeDtypeStruct((N, HID), jnp.float32),
        ],
    )(h, hist, W_emb, b_emb, W_gcn)


@functools.partial(
    pl.kernel,
    out_type=jax.ShapeDtypeStruct((NC, N, HID), jnp.float32),
    mesh=_mesh,
    compiler_params=_sc_params,
    scratch_types=[
        pltpu.VMEM_SHARED((N, HID), jnp.float32),
        pltpu.VMEM((4, 128), jnp.int32),
        pltpu.VMEM((4, 128), jnp.int32),
        pltpu.VMEM((512, HID), jnp.float32),
    ] + [pltpu.SemaphoreType.DMA] * 16,
)
def _gs_call(m_hbm, src2d, dst2d, zeros_hbm, agg_out,
             shared, sidx, didx, rows, *sems):
    gsem, ssem, sisem, disem = sems[0:4], sems[4:8], sems[8:12], sems[12:16]
    c = lax.axis_index("c")
    s = lax.axis_index("s")
    wid = s * NC + c
    pltpu.sync_copy(zeros_hbm.at[pl.ds(s * ZPT, ZPT)],
                    shared.at[pl.ds(s * ZPT, ZPT)])
    plsc.subcore_barrier()

    def row_of(k):
        return k * NW + wid

    def wait_iload(edge2d, buf, sem):
        pltpu.make_async_copy(edge2d.at[pl.ds(0, 1)],
                              buf, sem).wait()

    for b in range(4):
        pltpu.async_copy(src2d.at[pl.ds(row_of(b), 1)],
                         sidx.at[pl.ds(b, 1)], sisem[b])
        pltpu.async_copy(dst2d.at[pl.ds(row_of(b), 1)],
                         didx.at[pl.ds(b, 1)], disem[b])
    for b in range(4):
        wait_iload(src2d, sidx.at[pl.ds(b, 1)], sisem[b])
        pltpu.async_copy(m_hbm.at[sidx.at[b]],
                         rows.at[pl.ds(b * 128, 128)], gsem[b])

    def step(k, b):
        pred = row_of(k) < ER
        predn = row_of(k + 4) < ER

        @pl.when(pred)
        def _():
            pltpu.make_async_copy(m_hbm.at[sidx.at[b]],
                                  rows.at[pl.ds(b * 128, 128)],
                                  gsem[b]).wait()

        @pl.when(predn)
        def _():
            pltpu.async_copy(src2d.at[pl.ds(row_of(k + 4), 1)],
                             sidx.at[pl.ds(b, 1)], sisem[b])

        @pl.when(pred)
        def _():
            wait_iload(dst2d, didx.at[pl.ds(b, 1)], disem[b])
            pltpu.async_copy(rows.at[pl.ds(b * 128, 128)],
                             shared.at[didx.at[b]], ssem[b], add=True)
            pltpu.make_async_copy(rows.at[pl.ds(b * 128, 128)],
                                  shared.at[didx.at[b]], ssem[b]).wait()

        @pl.when(predn)
        def _():
            pltpu.async_copy(dst2d.at[pl.ds(row_of(k + 4), 1)],
                             didx.at[pl.ds(b, 1)], disem[b])
            wait_iload(src2d, sidx.at[pl.ds(b, 1)], sisem[b])
            pltpu.async_copy(m_hbm.at[sidx.at[b]],
                             rows.at[pl.ds(b * 128, 128)], gsem[b])

    def body(kk, carry):
        for b in range(4):
            step(4 * kk + b, b)
        return carry

    lax.fori_loop(0, S3_STEPS // 4, body, 0)

    plsc.subcore_barrier()
    pltpu.sync_copy(shared.at[pl.ds(s * ZPT, ZPT)],
                    agg_out.at[c, pl.ds(s * ZPT, ZPT)])


def _final_body(h1_ref, agg_ref, hist_ref, bg_ref, g_ref, beta_ref,
                hgp_ref):
    agg = agg_ref[0] + agg_ref[1]
    deg = hist_ref[0][:, 0:1]
    agg = agg * lax.rsqrt(jnp.maximum(deg, 1.0)) + bg_ref[...]
    xx = agg * g_ref[...] + beta_ref[...]
    xx = jnp.where(xx > 0, xx, 0.01 * xx)
    xx = h1_ref[...] + xx
    s0 = jnp.sum(xx[:NODES_PER_GRAPH], axis=0, keepdims=True)
    s1 = jnp.sum(xx[NODES_PER_GRAPH:], axis=0, keepdims=True)
    z = jnp.zeros((6, HID), jnp.float32)
    hgp_ref[...] = jnp.concatenate([s0, s1, z], axis=0) * (1.0 / NODES_PER_GRAPH)


def _final_call(h1, aggp, hist, bg, g, beta):
    blk = 2 * NODES_PER_GRAPH
    return pl.pallas_call(
        _final_body,
        grid=(N // blk,),
        in_specs=[
            pl.BlockSpec((blk, HID), lambda i: (i, 0)),
            pl.BlockSpec((NC, blk, HID), lambda i: (0, i, 0)),
            pl.BlockSpec((1, blk, HWID), lambda i: (1, i, 0)),
            pl.BlockSpec((1, HID), lambda i: (0, 0)),
            pl.BlockSpec((1, HID), lambda i: (0, 0)),
            pl.BlockSpec((1, HID), lambda i: (0, 0)),
        ],
        out_specs=pl.BlockSpec((8, HID), lambda i: (i, 0)),
        out_shape=jax.ShapeDtypeStruct((8 * (N // blk), HID), jnp.float32),
    )(h1, aggp, hist, bg, g, beta)


def _mlp_body(hg_ref, w0, b0, w1, b1, w2, b2, out_ref):
    y = jnp.maximum(jnp.dot(hg_ref[...], w0[...],
                            preferred_element_type=jnp.float32) + b0[...], 0.0)
    y = jnp.maximum(jnp.dot(y, w1[...],
                            preferred_element_type=jnp.float32) + b1[...], 0.0)
    out_ref[...] = jnp.dot(y, w2[...],
                           preferred_element_type=jnp.float32) + b2[...]


def _mlp_call(hg, W0, b0, W1, b1, W2, b2):
    return pl.pallas_call(
        _mlp_body,
        out_shape=jax.ShapeDtypeStruct((NUM_GRAPHS, 2), jnp.float32),
    )(hg, W0, b0, W1, b1, W2, b2)


def kernel(h, edge_index, id, e, W_emb, b_emb, W_gcn, b_gcn, gamma, beta,
           W0, b0, W1, b1, W2, b2):
    src2d = edge_index[0].reshape(ER, 128)
    dst2d = edge_index[1].reshape(ER, 128)
    ones_h = jnp.ones((128, HWID), jnp.float32)
    zeros_h = jnp.zeros((N, HWID), jnp.float32)
    zeros_a = jnp.zeros((N, HID), jnp.float32)

    hist = _hist_call(src2d, dst2d, ones_h, zeros_h)
    h1, m = _dense_call(h, hist, W_emb, b_emb.reshape(1, HID), W_gcn)
    aggp = _gs_call(m, src2d, dst2d, zeros_a)

    g = (gamma / jnp.sqrt(1.0 + 1e-5)).reshape(1, HID)
    hgp = _final_call(h1, aggp, hist, b_gcn.reshape(1, HID), g,
                      beta.reshape(1, HID))
    hg = hgp.reshape(N // (2 * NODES_PER_GRAPH), 8, HID)[:, :2]
    hg = hg.reshape(NUM_GRAPHS, HID)
    return _mlp_call(hg, W0, b0.reshape(1, 16), W1, b1.reshape(1, 8),
                     W2, b2.reshape(1, 2))
